# baseline (speedup 1.0000x reference)
import jax
import jax.numpy as jnp
from jax.experimental import pallas as pl
from jax.experimental.pallas import tpu as pltpu

M_BLK = 10000


KSPLIT = 4


def _matmul_body(*refs):
    x_refs = refs[:KSPLIT]
    w_ref = refs[KSPLIT]
    o_ref = refs[KSPLIT + 1]
    kc = w_ref.shape[0] // KSPLIT
    acc = jnp.dot(x_refs[0][...], w_ref[pl.ds(0, kc), :],
                  preferred_element_type=jnp.float32)
    for s in range(1, KSPLIT):
        acc += jnp.dot(x_refs[s][...], w_ref[pl.ds(s * kc, kc), :],
                       preferred_element_type=jnp.float32)
    o_ref[...] = acc


def _content_matmul(item_content, W):
    M, K = item_content.shape
    N = W.shape[1]
    kc = K // KSPLIT
    grid = (M // M_BLK,)
    x_specs = [
        pl.BlockSpec((M_BLK, kc), lambda i, s=s: (i, s))
        for s in range(KSPLIT)
    ]
    return pl.pallas_call(
        _matmul_body,
        grid=grid,
        in_specs=x_specs + [pl.BlockSpec((K, N), lambda i: (0, 0))],
        out_specs=pl.BlockSpec((M_BLK, N), lambda i: (i, 0)),
        out_shape=jax.ShapeDtypeStruct((M, N), jnp.float32),
        compiler_params=pltpu.CompilerParams(
            dimension_semantics=("arbitrary",),
        ),
    )(*([item_content] * KSPLIT), W)


def kernel(P, Q, PQ2, item_content, W):
    item_emb2 = _content_matmul(item_content, W)
    return (P, Q, PQ2, item_emb2, W)


# speedup vs baseline: 1.0008x; 1.0008x over previous
import jax
import jax.numpy as jnp
from jax.experimental import pallas as pl
from jax.experimental.pallas import tpu as pltpu

M_BLK = 10000


def _matmul_body(x_ref, w_ref, o_ref):
    o_ref[...] = jnp.dot(x_ref[...], w_ref[...],
                         preferred_element_type=jnp.float32)


def _content_matmul(item_content, W):
    M, K = item_content.shape
    N = W.shape[1]
    grid = (M // M_BLK,)
    return pl.pallas_call(
        _matmul_body,
        grid=grid,
        in_specs=[
            pl.BlockSpec((M_BLK, K), lambda i: (i, 0)),
            pl.BlockSpec((K, N), lambda i: (0, 0)),
        ],
        out_specs=pl.BlockSpec((M_BLK, N), lambda i: (i, 0)),
        out_shape=jax.ShapeDtypeStruct((M, N), jnp.float32),
        compiler_params=pltpu.CompilerParams(
            dimension_semantics=("arbitrary",),
            vmem_limit_bytes=110 * 1024 * 1024,
        ),
    )(item_content, W)


def kernel(P, Q, PQ2, item_content, W):
    item_emb2 = _content_matmul(item_content, W)
    return (P, Q, PQ2, item_emb2, W)
